# baseline (speedup 1.0000x reference)
import jax
import jax.numpy as jnp
from jax.experimental import pallas as pl
from jax.experimental.pallas import tpu as pltpu

_T = 13
_COU_EMBED = 32


def _main_body(sidx_ref,
               sd_ref, ed_ref, pt_ref, dt_ref, m_ref, a_ref,
               sdT_ref, edT_ref, v_ref, vpt_ref, vdt_ref, vft_ref,
               vnum_ref, dm_ref, wn_ref, we_ref,
               node_out_ref, edge_out_ref):
    b = pl.program_id(0)
    t = pl.program_id(1)
    s = sidx_ref[b, t]
    n = node_out_ref.shape[2]

    we = we_ref[...]
    m = m_ref[0, 0]
    acc = (sd_ref[0] * m) * we[0:1, :]
    acc += (ed_ref[0] * m) * we[1:2, :]
    acc += (pt_ref[0] * m) * we[2:3, :]
    acc += (dt_ref[0] * m) * we[3:4, :]
    acc += a_ref[0, 0] * we[4:5, :]
    edge_out_ref[0, 0] = acc

    ohl = (jax.lax.broadcasted_iota(jnp.int32, (1, n), 1) == s
           ).astype(jnp.float32)
    ohs = (jax.lax.broadcasted_iota(jnp.int32, (n, 1), 0) == s
           ).astype(jnp.float32)
    t_c = jnp.sum(vft_ref[0] * ohs)
    ed_row = jnp.sum(edT_ref[0] * ohl, axis=1, keepdims=True)
    sd_row = jnp.sum(sdT_ref[0] * ohl, axis=1, keepdims=True)

    dm = dm_ref[0, 0]
    v = v_ref[0]
    wn = wn_ref[...]
    node = (v[:, 0:1] * dm) * wn[0:1, :]
    node += (v[:, 1:2] * dm) * wn[1:2, :]
    node += (v[:, 2:3] * dm) * wn[2:3, :]
    node += ((vpt_ref[0] - t_c) * dm) * wn[3:4, :]
    node += ((t_c - vdt_ref[0]) * dm) * wn[4:5, :]
    node += (ed_row * dm * dm) * wn[5:6, :]
    node += (sd_row * dm * dm) * wn[6:7, :]
    node += (vnum_ref[0, 0] * dm) * wn[7:8, :]
    node_out_ref[0, 0] = node


def _embed_body(ids_ref, table_ref, cou_ref, out_ref):
    del ids_ref
    out_ref[0, :, 0:_COU_EMBED] = table_ref[0]
    out_ref[0, :, _COU_EMBED:_COU_EMBED + 3] = cou_ref[0][:, 1:4]


def kernel(E_ed, V, V_reach_mask, V_pt, E_sd, V_ft, start_idx, V_dt, V_num,
           E_mask, V_dispatch_mask, E_pt_dif, E_dt_dif, cou, A, W_node,
           W_edge, cou_table):
    del V_reach_mask
    B, N, _ = V.shape
    T = start_idx.shape[1]
    NN = N * N
    H = W_node.shape[1]
    f32 = jnp.float32

    sidx = start_idx.astype(jnp.int32)
    sd_r = E_sd.reshape(B, NN, 1)
    ed_r = E_ed.reshape(B, NN, 1)
    pt_r = E_pt_dif.reshape(B, NN, 1)
    dt_r = E_dt_dif.reshape(B, NN, 1)
    m_r = E_mask.reshape(B, T, NN, 1)
    a_r = A.reshape(B, T, NN, 1)
    sdT = E_sd.transpose(0, 2, 1)
    edT = E_ed.transpose(0, 2, 1)
    vpt_r = V_pt.reshape(B, N, 1)
    vdt_r = V_dt.reshape(B, N, 1)
    vft_r = V_ft.reshape(B, N, 1)
    vnum_r = V_num.reshape(B, T, N, 1)
    dm_r = V_dispatch_mask.reshape(B, T, N, 1)

    per_b3 = pl.BlockSpec((1, NN, 1), lambda b, t, s_ref: (b, 0, 0))
    per_bt4 = pl.BlockSpec((1, 1, NN, 1), lambda b, t, s_ref: (b, t, 0, 0))
    per_bT = pl.BlockSpec((1, N, N), lambda b, t, s_ref: (b, 0, 0))
    per_bn = pl.BlockSpec((1, N, 1), lambda b, t, s_ref: (b, 0, 0))
    per_btn = pl.BlockSpec((1, 1, N, 1), lambda b, t, s_ref: (b, t, 0, 0))

    node_h, edge_r = pl.pallas_call(
        _main_body,
        grid_spec=pltpu.PrefetchScalarGridSpec(
            num_scalar_prefetch=1,
            grid=(B, T),
            in_specs=[
                per_b3, per_b3, per_b3, per_b3,
                per_bt4, per_bt4,
                per_bT, per_bT,
                pl.BlockSpec((1, N, 3), lambda b, t, s_ref: (b, 0, 0)),
                per_bn, per_bn, per_bn,
                per_btn, per_btn,
                pl.BlockSpec((8, H), lambda b, t, s_ref: (0, 0)),
                pl.BlockSpec((5, H), lambda b, t, s_ref: (0, 0)),
            ],
            out_specs=[
                pl.BlockSpec((1, 1, N, H), lambda b, t, s_ref: (b, t, 0, 0)),
                pl.BlockSpec((1, 1, NN, H), lambda b, t, s_ref: (b, t, 0, 0)),
            ],
        ),
        out_shape=[
            jax.ShapeDtypeStruct((B, T, N, H), f32),
            jax.ShapeDtypeStruct((B, T, NN, H), f32),
        ],
    )(sidx, sd_r, ed_r, pt_r, dt_r, m_r, a_r, sdT, edT, V,
      vpt_r, vdt_r, vft_r, vnum_r, dm_r, W_node, W_edge)

    edge_h = edge_r.reshape(B, T, N, N, H)

    ids = cou[:, 0].astype(jnp.int32)
    E = _COU_EMBED
    table3 = cou_table.reshape(cou_table.shape[0], 1, E)
    cou3 = cou.reshape(B, 1, 4)
    embed_b = pl.pallas_call(
        _embed_body,
        grid_spec=pltpu.PrefetchScalarGridSpec(
            num_scalar_prefetch=1,
            grid=(B,),
            in_specs=[
                pl.BlockSpec((1, 1, E), lambda b, ids_ref: (ids_ref[b], 0, 0)),
                pl.BlockSpec((1, 1, 4), lambda b, ids_ref: (b, 0, 0)),
            ],
            out_specs=pl.BlockSpec((1, 1, E + 3), lambda b, ids_ref: (b, 0, 0)),
        ),
        out_shape=jax.ShapeDtypeStruct((B, 1, E + 3), f32),
    )(ids, table3, cou3)
    embed_cou = jnp.broadcast_to(embed_b, (B, _T, E + 3)).reshape(B * _T, E + 3)

    return (node_h, edge_h, embed_cou)


# speedup vs baseline: 1.7080x; 1.7080x over previous
import jax
import jax.numpy as jnp
from jax.experimental import pallas as pl
from jax.experimental.pallas import tpu as pltpu

_T = 13
_COU_EMBED = 32
_DN = (((0,), (0,)), ((), ()))


def _main_body(sidx_ref,
               sd_ref, ed_ref, pt_ref, dt_ref, m_ref, a_ref,
               sdn_ref, edn_ref, vT_ref, vpt_ref, vdt_ref, vft_ref,
               vnum_ref, dm_ref, wn_ref, we_ref,
               node_out_ref, edge_out_ref):
    b = pl.program_id(0)
    t = pl.program_id(1)
    s = sidx_ref[b, t]
    n = node_out_ref.shape[2]
    f32 = jnp.float32

    m = m_ref[0, 0]
    x = jnp.concatenate([
        sd_ref[0] * m,
        ed_ref[0] * m,
        pt_ref[0] * m,
        dt_ref[0] * m,
        a_ref[0, 0],
    ], axis=0)
    edge_out_ref[0, 0] = jax.lax.dot_general(
        x, we_ref[...], _DN, preferred_element_type=f32)

    ohs = (jax.lax.broadcasted_iota(jnp.int32, (n, 1), 0) == s).astype(f32)
    ohl = (jax.lax.broadcasted_iota(jnp.int32, (1, n), 1) == s).astype(f32)
    ed_row = jnp.sum(edn_ref[0] * ohs, axis=0, keepdims=True)
    sd_row = jnp.sum(sdn_ref[0] * ohs, axis=0, keepdims=True)
    t_c = jnp.sum(vft_ref[0] * ohl)
    dm = dm_ref[0, 0]
    xn = jnp.concatenate([
        vT_ref[0],
        vpt_ref[0] - t_c,
        t_c - vdt_ref[0],
        ed_row * dm,
        sd_row * dm,
        vnum_ref[0, 0],
    ], axis=0) * dm
    node_out_ref[0, 0] = jax.lax.dot_general(
        xn, wn_ref[...], _DN, preferred_element_type=f32)


def _embed_body(ids_ref, table_ref, cou_ref, out_ref):
    del ids_ref
    out_ref[0, :, 0:_COU_EMBED] = table_ref[0]
    out_ref[0, :, _COU_EMBED:_COU_EMBED + 3] = cou_ref[0][:, 1:4]


def kernel(E_ed, V, V_reach_mask, V_pt, E_sd, V_ft, start_idx, V_dt, V_num,
           E_mask, V_dispatch_mask, E_pt_dif, E_dt_dif, cou, A, W_node,
           W_edge, cou_table):
    del V_reach_mask
    B, N, _ = V.shape
    T = start_idx.shape[1]
    NN = N * N
    H = W_node.shape[1]
    f32 = jnp.float32

    sidx = start_idx.astype(jnp.int32)
    sd_f = E_sd.reshape(B, 1, NN)
    ed_f = E_ed.reshape(B, 1, NN)
    pt_f = E_pt_dif.reshape(B, 1, NN)
    dt_f = E_dt_dif.reshape(B, 1, NN)
    m_f = E_mask.reshape(B, T, 1, NN)
    a_f = A.reshape(B, T, 1, NN)
    vT = V.transpose(0, 2, 1)
    vpt_f = V_pt.reshape(B, 1, N)
    vdt_f = V_dt.reshape(B, 1, N)
    vft_f = V_ft.reshape(B, 1, N)
    vnum_f = V_num.reshape(B, T, 1, N)
    dm_f = V_dispatch_mask.reshape(B, T, 1, N)

    per_bf = pl.BlockSpec((1, 1, NN), lambda b, t, s_ref: (b, 0, 0))
    per_btf = pl.BlockSpec((1, 1, 1, NN), lambda b, t, s_ref: (b, t, 0, 0))
    per_bnat = pl.BlockSpec((1, N, N), lambda b, t, s_ref: (b, 0, 0))
    per_bn = pl.BlockSpec((1, 1, N), lambda b, t, s_ref: (b, 0, 0))
    per_btn = pl.BlockSpec((1, 1, 1, N), lambda b, t, s_ref: (b, t, 0, 0))

    node_h, edge_r = pl.pallas_call(
        _main_body,
        grid_spec=pltpu.PrefetchScalarGridSpec(
            num_scalar_prefetch=1,
            grid=(B, T),
            in_specs=[
                per_bf, per_bf, per_bf, per_bf,
                per_btf, per_btf,
                per_bnat, per_bnat,
                pl.BlockSpec((1, 3, N), lambda b, t, s_ref: (b, 0, 0)),
                per_bn, per_bn, per_bn,
                per_btn, per_btn,
                pl.BlockSpec((8, H), lambda b, t, s_ref: (0, 0)),
                pl.BlockSpec((5, H), lambda b, t, s_ref: (0, 0)),
            ],
            out_specs=[
                pl.BlockSpec((1, 1, N, H), lambda b, t, s_ref: (b, t, 0, 0)),
                pl.BlockSpec((1, 1, NN, H), lambda b, t, s_ref: (b, t, 0, 0)),
            ],
        ),
        out_shape=[
            jax.ShapeDtypeStruct((B, T, N, H), f32),
            jax.ShapeDtypeStruct((B, T, NN, H), f32),
        ],
    )(sidx, sd_f, ed_f, pt_f, dt_f, m_f, a_f, E_sd, E_ed, vT,
      vpt_f, vdt_f, vft_f, vnum_f, dm_f, W_node, W_edge)

    edge_h = edge_r.reshape(B, T, N, N, H)

    ids = cou[:, 0].astype(jnp.int32)
    E = _COU_EMBED
    table3 = cou_table.reshape(cou_table.shape[0], 1, E)
    cou3 = cou.reshape(B, 1, 4)
    embed_b = pl.pallas_call(
        _embed_body,
        grid_spec=pltpu.PrefetchScalarGridSpec(
            num_scalar_prefetch=1,
            grid=(B,),
            in_specs=[
                pl.BlockSpec((1, 1, E), lambda b, ids_ref: (ids_ref[b], 0, 0)),
                pl.BlockSpec((1, 1, 4), lambda b, ids_ref: (b, 0, 0)),
            ],
            out_specs=pl.BlockSpec((1, 1, E + 3), lambda b, ids_ref: (b, 0, 0)),
        ),
        out_shape=jax.ShapeDtypeStruct((B, 1, E + 3), f32),
    )(ids, table3, cou3)
    embed_cou = jnp.broadcast_to(embed_b, (B, _T, E + 3)).reshape(B * _T, E + 3)

    return (node_h, edge_h, embed_cou)


# speedup vs baseline: 2.8982x; 1.6968x over previous
import jax
import jax.numpy as jnp
from jax.experimental import pallas as pl
from jax.experimental.pallas import tpu as pltpu

_T = 13
_COU_EMBED = 32
_DN = (((0,), (0,)), ((), ()))


def _main_body(sidx_ref,
               sd_ref, ed_ref, pt_ref, dt_ref, m_ref, a_ref,
               sdn_ref, edn_ref, vT_ref, vpt_ref, vdt_ref, vft_ref,
               vnum_ref, dm_ref, wn_ref, we_ref,
               node_out_ref, edge_out_ref):
    b = pl.program_id(0)
    n = node_out_ref.shape[2]
    f32 = jnp.float32

    sd = sd_ref[0]
    ed = ed_ref[0]
    pt = pt_ref[0]
    dt = dt_ref[0]
    we = we_ref[...]
    wn = wn_ref[...]
    sdn = sdn_ref[0]
    edn = edn_ref[0]
    vT = vT_ref[0]
    vpt = vpt_ref[0]
    vdt = vdt_ref[0]
    vft = vft_ref[0]
    iota_s = jax.lax.broadcasted_iota(jnp.int32, (n, 1), 0)
    iota_l = jax.lax.broadcasted_iota(jnp.int32, (1, n), 1)

    for t in range(_T):
        m = m_ref[0, t]
        x = jnp.concatenate(
            [sd * m, ed * m, pt * m, dt * m, a_ref[0, t]], axis=0)
        edge_out_ref[0, t] = jax.lax.dot_general(
            x, we, _DN, preferred_element_type=f32)

        s = sidx_ref[b, t]
        ohs = (iota_s == s).astype(f32)
        ohl = (iota_l == s).astype(f32)
        ed_row = jnp.sum(edn * ohs, axis=0, keepdims=True)
        sd_row = jnp.sum(sdn * ohs, axis=0, keepdims=True)
        t_c = jnp.sum(vft * ohl)
        dm = dm_ref[0, t]
        xn = jnp.concatenate([
            vT,
            vpt - t_c,
            t_c - vdt,
            ed_row * dm,
            sd_row * dm,
            vnum_ref[0, t],
        ], axis=0) * dm
        node_out_ref[0, t] = jax.lax.dot_general(
            xn, wn, _DN, preferred_element_type=f32)


def _embed_body(ids_ref, table_ref, cou_ref, out_ref):
    del ids_ref
    out_ref[0, :, 0:_COU_EMBED] = table_ref[0]
    out_ref[0, :, _COU_EMBED:_COU_EMBED + 3] = cou_ref[0][:, 1:4]


def kernel(E_ed, V, V_reach_mask, V_pt, E_sd, V_ft, start_idx, V_dt, V_num,
           E_mask, V_dispatch_mask, E_pt_dif, E_dt_dif, cou, A, W_node,
           W_edge, cou_table):
    del V_reach_mask
    B, N, _ = V.shape
    T = start_idx.shape[1]
    NN = N * N
    H = W_node.shape[1]
    f32 = jnp.float32

    sidx = start_idx.astype(jnp.int32)
    sd_f = E_sd.reshape(B, 1, NN)
    ed_f = E_ed.reshape(B, 1, NN)
    pt_f = E_pt_dif.reshape(B, 1, NN)
    dt_f = E_dt_dif.reshape(B, 1, NN)
    m_f = E_mask.reshape(B, T, 1, NN)
    a_f = A.reshape(B, T, 1, NN)
    vT = V.transpose(0, 2, 1)
    vpt_f = V_pt.reshape(B, 1, N)
    vdt_f = V_dt.reshape(B, 1, N)
    vft_f = V_ft.reshape(B, 1, N)
    vnum_f = V_num.reshape(B, T, 1, N)
    dm_f = V_dispatch_mask.reshape(B, T, 1, N)

    per_bf = pl.BlockSpec((1, 1, NN), lambda b, s_ref: (b, 0, 0))
    per_btf = pl.BlockSpec((1, T, 1, NN), lambda b, s_ref: (b, 0, 0, 0))
    per_bnat = pl.BlockSpec((1, N, N), lambda b, s_ref: (b, 0, 0))
    per_bn = pl.BlockSpec((1, 1, N), lambda b, s_ref: (b, 0, 0))
    per_btn = pl.BlockSpec((1, T, 1, N), lambda b, s_ref: (b, 0, 0, 0))

    node_h, edge_r = pl.pallas_call(
        _main_body,
        grid_spec=pltpu.PrefetchScalarGridSpec(
            num_scalar_prefetch=1,
            grid=(B,),
            in_specs=[
                per_bf, per_bf, per_bf, per_bf,
                per_btf, per_btf,
                per_bnat, per_bnat,
                pl.BlockSpec((1, 3, N), lambda b, s_ref: (b, 0, 0)),
                per_bn, per_bn, per_bn,
                per_btn, per_btn,
                pl.BlockSpec((8, H), lambda b, s_ref: (0, 0)),
                pl.BlockSpec((5, H), lambda b, s_ref: (0, 0)),
            ],
            out_specs=[
                pl.BlockSpec((1, T, N, H), lambda b, s_ref: (b, 0, 0, 0)),
                pl.BlockSpec((1, T, NN, H), lambda b, s_ref: (b, 0, 0, 0)),
            ],
        ),
        out_shape=[
            jax.ShapeDtypeStruct((B, T, N, H), f32),
            jax.ShapeDtypeStruct((B, T, NN, H), f32),
        ],
    )(sidx, sd_f, ed_f, pt_f, dt_f, m_f, a_f, E_sd, E_ed, vT,
      vpt_f, vdt_f, vft_f, vnum_f, dm_f, W_node, W_edge)

    edge_h = edge_r.reshape(B, T, N, N, H)

    ids = cou[:, 0].astype(jnp.int32)
    E = _COU_EMBED
    table3 = cou_table.reshape(cou_table.shape[0], 1, E)
    cou3 = cou.reshape(B, 1, 4)
    embed_b = pl.pallas_call(
        _embed_body,
        grid_spec=pltpu.PrefetchScalarGridSpec(
            num_scalar_prefetch=1,
            grid=(B,),
            in_specs=[
                pl.BlockSpec((1, 1, E), lambda b, ids_ref: (ids_ref[b], 0, 0)),
                pl.BlockSpec((1, 1, 4), lambda b, ids_ref: (b, 0, 0)),
            ],
            out_specs=pl.BlockSpec((1, 1, E + 3), lambda b, ids_ref: (b, 0, 0)),
        ),
        out_shape=jax.ShapeDtypeStruct((B, 1, E + 3), f32),
    )(ids, table3, cou3)
    embed_cou = jnp.broadcast_to(embed_b, (B, _T, E + 3)).reshape(B * _T, E + 3)

    return (node_h, edge_h, embed_cou)
